# baseline (speedup 1.0000x reference)
import functools

import jax
import jax.numpy as jnp
from jax.experimental import pallas as pl
from jax.experimental.pallas import tpu as pltpu

N_PEAKS_ = 64
DIM_ = 128


def _cond_body(nrows, x_ref, s_ref, b_ref, o_ref):
    off = (pl.program_id(0) * nrows) % N_PEAKS_
    s = s_ref[pl.ds(off, nrows), :][:, :, None]
    b = b_ref[pl.ds(off, nrows), :][:, :, None]
    o_ref[...] = x_ref[...] * s + b


def kernel(x, queries, W_scale, W_bias):
    del queries
    rows, dim, t = x.shape
    R = 16
    grid = (rows // R,)

    out = pl.pallas_call(
        functools.partial(_cond_body, R),
        grid=grid,
        in_specs=[
            pl.BlockSpec((R, dim, t), lambda i: (i, 0, 0)),
            pl.BlockSpec((N_PEAKS_, dim), lambda i: (0, 0)),
            pl.BlockSpec((N_PEAKS_, dim), lambda i: (0, 0)),
        ],
        out_specs=pl.BlockSpec((R, dim, t), lambda i: (i, 0, 0)),
        out_shape=jax.ShapeDtypeStruct(x.shape, x.dtype),
        compiler_params=pltpu.CompilerParams(
            dimension_semantics=("parallel",),
        ),
    )(x, W_scale, W_bias)
    return out


# speedup vs baseline: 1.0243x; 1.0243x over previous
import functools

import jax
import jax.numpy as jnp
from jax.experimental import pallas as pl
from jax.experimental.pallas import tpu as pltpu

N_PEAKS_ = 64
DIM_ = 128


def _cond_body(x_ref, s_ref, b_ref, o_ref):
    s = s_ref[...][:, :, None]
    b = b_ref[...][:, :, None]
    o_ref[...] = x_ref[...] * s + b


def kernel(x, queries, W_scale, W_bias):
    del queries
    rows, dim, t = x.shape
    R = 64
    TS = 2
    tb = t // TS
    grid = (rows // R, TS)

    out = pl.pallas_call(
        _cond_body,
        grid=grid,
        in_specs=[
            pl.BlockSpec((R, dim, tb), lambda i, j: (i, 0, j)),
            pl.BlockSpec((N_PEAKS_, dim), lambda i, j: (0, 0)),
            pl.BlockSpec((N_PEAKS_, dim), lambda i, j: (0, 0)),
        ],
        out_specs=pl.BlockSpec((R, dim, tb), lambda i, j: (i, 0, j)),
        out_shape=jax.ShapeDtypeStruct(x.shape, x.dtype),
        compiler_params=pltpu.CompilerParams(
            dimension_semantics=("parallel", "parallel"),
        ),
    )(x, W_scale, W_bias)
    return out


# speedup vs baseline: 1.2172x; 1.1884x over previous
import functools

import jax
import jax.numpy as jnp
from jax.experimental import pallas as pl
from jax.experimental.pallas import tpu as pltpu

N_PEAKS_ = 64
DIM_ = 128


def _cond_body(x_ref, s_ref, b_ref, o_ref):
    s = s_ref[...][:, :, None]
    b = b_ref[...][:, :, None]
    o_ref[...] = x_ref[...] * s + b


def kernel(x, queries, W_scale, W_bias):
    del queries
    rows, dim, t = x.shape
    R = 64
    grid = (rows // R,)

    out = pl.pallas_call(
        _cond_body,
        grid=grid,
        in_specs=[
            pl.BlockSpec((R, dim, t), lambda i: (i, 0, 0)),
            pl.BlockSpec((N_PEAKS_, dim), lambda i: (0, 0)),
            pl.BlockSpec((N_PEAKS_, dim), lambda i: (0, 0)),
        ],
        out_specs=pl.BlockSpec((R, dim, t), lambda i: (i, 0, 0)),
        out_shape=jax.ShapeDtypeStruct(x.shape, x.dtype),
        compiler_params=pltpu.CompilerParams(
            dimension_semantics=("parallel",),
        ),
    )(x, W_scale, W_bias)
    return out
